# baseline (speedup 1.0000x reference)
import jax
import jax.numpy as jnp
from jax.experimental import pallas as pl
from jax.experimental.pallas import tpu as pltpu

_BM = 400


def _support_kernel(feature_ref, w_ref, b_ref, out_ref):
    sup = jnp.dot(feature_ref[...], w_ref[...],
                  preferred_element_type=jnp.float32) + b_ref[...]
    out_ref[...] = sup.astype(jnp.bfloat16)


def _agg_kernel(adj_ref, support_ref, out_ref):
    a = adj_ref[...].astype(jnp.bfloat16)
    h = jnp.dot(a, support_ref[...], preferred_element_type=jnp.float32)
    out_ref[...] = jnp.tanh(h)


def kernel(feature, adj, W, b):
    n, d = feature.shape
    b2 = b.reshape(1, d)
    support = pl.pallas_call(
        _support_kernel,
        out_shape=jax.ShapeDtypeStruct((n, d), jnp.bfloat16),
    )(feature, W, b2)
    return pl.pallas_call(
        _agg_kernel,
        grid=(n // _BM,),
        in_specs=[
            pl.BlockSpec((_BM, n), lambda i: (i, 0)),
            pl.BlockSpec((n, d), lambda i: (0, 0)),
        ],
        out_specs=pl.BlockSpec((_BM, d), lambda i: (i, 0)),
        out_shape=jax.ShapeDtypeStruct((n, d), jnp.float32),
        compiler_params=pltpu.CompilerParams(
            dimension_semantics=("arbitrary",),
        ),
    )(adj, support)


# speedup vs baseline: 1.0330x; 1.0330x over previous
import jax
import jax.numpy as jnp
from jax.experimental import pallas as pl
from jax.experimental.pallas import tpu as pltpu

_BM = 400


def _agg_kernel(feature_ref, adj_ref, w_ref, b_ref, out_ref, support_ref):
    @pl.when(pl.program_id(0) == 0)
    def _():
        support_ref[...] = jnp.dot(
            feature_ref[...], w_ref[...],
            preferred_element_type=jnp.float32) + b_ref[...]

    h = jnp.dot(adj_ref[...], support_ref[...],
                preferred_element_type=jnp.float32)
    out_ref[...] = jnp.tanh(h)


def kernel(feature, adj, W, b):
    n, d = feature.shape
    b2 = b.reshape(1, d)
    return pl.pallas_call(
        _agg_kernel,
        grid=(n // _BM,),
        in_specs=[
            pl.BlockSpec((n, d), lambda i: (0, 0)),
            pl.BlockSpec((_BM, n), lambda i: (i, 0)),
            pl.BlockSpec((d, d), lambda i: (0, 0)),
            pl.BlockSpec((1, d), lambda i: (0, 0)),
        ],
        out_specs=pl.BlockSpec((_BM, d), lambda i: (i, 0)),
        out_shape=jax.ShapeDtypeStruct((n, d), jnp.float32),
        scratch_shapes=[pltpu.VMEM((n, d), jnp.float32)],
        compiler_params=pltpu.CompilerParams(
            dimension_semantics=("arbitrary",),
        ),
    )(feature, adj, W, b2)
